# baseline (speedup 1.0000x reference)
import functools

import jax
import jax.numpy as jnp
from jax import lax
from jax.experimental import pallas as pl
from jax.experimental.pallas import tpu as pltpu

B = 8
P = 128
NI = 4096
NU = 16384
F = 200
LL = 49
FC = 256
DC = 64
DI = 64
PC = 16
NCH = P // PC


def _compact_body(up_ref, gi_t, pi_t, pos_ref, gir_ref, pir_ref):
    mrow = up_ref[0] > 0.0
    cs = mrow.astype(jnp.int32)
    sh = 1
    while sh < NI:
        z = jnp.zeros((1, sh), jnp.int32)
        cs = cs + jnp.concatenate([z, cs[:, :NI - sh]], axis=1)
        sh *= 2
    kk = lax.broadcasted_iota(jnp.int32, (P, NI), 0)
    sel = (cs == (kk + 1)) & mrow
    a2 = jnp.where(sel, 1.0, 0.0)
    ii = lax.broadcasted_iota(jnp.int32, (P, NI), 1).astype(jnp.float32)
    pos = jnp.sum(a2 * ii, axis=1, keepdims=True)
    pos_ref[0] = pos.astype(jnp.int32)
    gir_ref[0] = jnp.dot(a2, gi_t[...], preferred_element_type=jnp.float32)
    pir_ref[0] = jnp.dot(a2, pi_t[...], preferred_element_type=jnp.float32)


def _main_body(pos_ref, u_ref, it_ref, *refs):
    fi_refs = refs[:PC]
    (gu_ref, giB_ref, piB_ref, gic_ref, pif_ref,
     wc0u, wc0i, bc0, wc1, bc1,
     wi0u, wi0iv, wi0ip, wi0ix, bi0, wi1, bi1,
     xui_ref, guo_ref, gio_ref, pio_ref, logit_scr) = refs[PC:]

    c = pl.program_id(1)
    gu = gu_ref[0]
    fi = jnp.concatenate([r[...] for r in fi_refs], axis=0)

    gu_c = jnp.dot(gu, wc0u[...], preferred_element_type=jnp.float32)
    t = lax.dot_general(fi, wc0i[...], (((2,), (0,)), ((), ())),
                        preferred_element_type=jnp.float32)
    t = jnp.maximum(t + gu_c[None] + bc0[...][None], 0.0)
    s = jnp.sum(t * wc1[...][None], axis=2) + bc1[0, 0]
    s = s - jnp.max(s, axis=1, keepdims=True)
    e = jnp.exp(s)
    w = e / jnp.sum(e, axis=1, keepdims=True)
    allx = jnp.sum(fi * w[:, :, None], axis=1)

    gi_c = gic_ref[0]
    pi_c = pif_ref[0, pl.ds(c * PC, PC), :]
    a = (jnp.dot(gu, wi0u[...], preferred_element_type=jnp.float32)
         + jnp.dot(gi_c, wi0iv[...], preferred_element_type=jnp.float32)
         + jnp.dot(pi_c, wi0ip[...], preferred_element_type=jnp.float32)
         + jnp.dot(allx, wi0ix[...], preferred_element_type=jnp.float32)
         + bi0[...])
    a = jnp.maximum(a, 0.0)
    logit = jnp.sum(a * wi1[...], axis=1, keepdims=True)
    logit_scr[pl.ds(c * PC, PC), :] = logit + bi1[0, 0]

    @pl.when(c == NCH - 1)
    def _finalize():
        lg = logit_scr[...]
        mx = jnp.max(lg, axis=0, keepdims=True)
        ee = jnp.exp(lg - mx)
        aw = ee / jnp.sum(ee, axis=0, keepdims=True)
        alla = jnp.sum(aw * pif_ref[0], axis=0, keepdims=True)
        gup = gu + alla
        gi_b = giB_ref[0]
        xui_ref[0] = jnp.sum(gup * gi_b).reshape(1, 1)
        guo_ref[0] = gu
        gio_ref[0] = gi_b
        pio_ref[0] = piB_ref[0]


@jax.jit
def kernel(user, item, user_pos, Gu, Gi, Pi, Fi,
           Wc0u, Wc0i, bc0, Wc1, bc1,
           Wi0u, Wi0iv, Wi0ip, Wi0ix, bi0, Wi1, bi1):
    up3 = user_pos.reshape(B, 1, NI)
    fi3 = Fi.reshape(NI, LL, FC)
    gu3 = Gu.reshape(NU, 1, F)
    gi3 = Gi.reshape(NI, 1, F)
    pi3 = Pi.reshape(NI, 1, F)
    bc0r = bc0.reshape(1, DC)
    bc1r = bc1.reshape(1, 1)
    bi0r = bi0.reshape(1, DI)
    bi1r = bi1.reshape(1, 1)

    pos, gi_rows, pi_rows = pl.pallas_call(
        _compact_body,
        grid=(B,),
        in_specs=[
            pl.BlockSpec((1, 1, NI), lambda b: (b, 0, 0)),
            pl.BlockSpec((NI, F), lambda b: (0, 0)),
            pl.BlockSpec((NI, F), lambda b: (0, 0)),
        ],
        out_specs=[
            pl.BlockSpec((1, P, 1), lambda b: (b, 0, 0)),
            pl.BlockSpec((1, P, F), lambda b: (b, 0, 0)),
            pl.BlockSpec((1, P, F), lambda b: (b, 0, 0)),
        ],
        out_shape=[
            jax.ShapeDtypeStruct((B, P, 1), jnp.int32),
            jax.ShapeDtypeStruct((B, P, F), jnp.float32),
            jax.ShapeDtypeStruct((B, P, F), jnp.float32),
        ],
    )(up3, Gi, Pi)
    pos2 = pos.reshape(B, P)

    def mk_fi_idx(j):
        def idx(b, c, pos_r, u_r, it_r):
            return (pos_r[b, c * PC + j], 0, 0)
        return idx

    fi_specs = [pl.BlockSpec((1, LL, FC), mk_fi_idx(j)) for j in range(PC)]
    wspec = lambda shape: pl.BlockSpec(shape, lambda b, c, *_: (0,) * len(shape))

    grid_spec = pltpu.PrefetchScalarGridSpec(
        num_scalar_prefetch=3,
        grid=(B, NCH),
        in_specs=fi_specs + [
            pl.BlockSpec((1, 1, F), lambda b, c, pos_r, u_r, it_r: (u_r[b], 0, 0)),
            pl.BlockSpec((1, 1, F), lambda b, c, pos_r, u_r, it_r: (it_r[b], 0, 0)),
            pl.BlockSpec((1, 1, F), lambda b, c, pos_r, u_r, it_r: (it_r[b], 0, 0)),
            pl.BlockSpec((1, PC, F), lambda b, c, *_: (b, c, 0)),
            pl.BlockSpec((1, P, F), lambda b, c, *_: (b, 0, 0)),
            wspec((F, DC)), wspec((FC, DC)), wspec((1, DC)),
            wspec((1, DC)), wspec((1, 1)),
            wspec((F, DI)), wspec((F, DI)), wspec((F, DI)),
            wspec((FC, DI)), wspec((1, DI)), wspec((1, DI)), wspec((1, 1)),
        ],
        out_specs=[
            pl.BlockSpec((1, 1, 1), lambda b, c, *_: (b, 0, 0)),
            pl.BlockSpec((1, 1, F), lambda b, c, *_: (b, 0, 0)),
            pl.BlockSpec((1, 1, F), lambda b, c, *_: (b, 0, 0)),
            pl.BlockSpec((1, 1, F), lambda b, c, *_: (b, 0, 0)),
        ],
        scratch_shapes=[pltpu.VMEM((P, 1), jnp.float32)],
    )

    xui3, guo, gio, pio = pl.pallas_call(
        _main_body,
        grid_spec=grid_spec,
        out_shape=[
            jax.ShapeDtypeStruct((B, 1, 1), jnp.float32),
            jax.ShapeDtypeStruct((B, 1, F), jnp.float32),
            jax.ShapeDtypeStruct((B, 1, F), jnp.float32),
            jax.ShapeDtypeStruct((B, 1, F), jnp.float32),
        ],
        compiler_params=pltpu.CompilerParams(
            dimension_semantics=("arbitrary", "arbitrary"),
        ),
    )(pos2, user.astype(jnp.int32), item.astype(jnp.int32),
      *([fi3] * PC),
      gu3, gi3, pi3, gi_rows, pi_rows,
      Wc0u, Wc0i, bc0r, Wc1, bc1r,
      Wi0u, Wi0iv, Wi0ip, Wi0ix, bi0r, Wi1, bi1r)

    return (xui3.reshape(B), guo.reshape(B, F), gio.reshape(B, F),
            pio.reshape(B, F))
